# baseline (speedup 1.0000x reference)
import functools

import jax
import jax.numpy as jnp
from jax import lax
from jax.experimental import pallas as pl
from jax.experimental.pallas import tpu as pltpu
from jax.experimental.pallas import tpu_sc as plsc

N_NODE = 50000
E_EDGE = 600000
D = 128

CH = 4096
CH_BITS = 12
NCHUNK = 13
NPAD = CH * NCHUNK
CHP = CH + 16
G = 128
BLK = 2048
NTILE = 16
NB = 19
EPT = BLK * NB
PAD_E = EPT * NTILE
SENT_DST = 1 << 29
ZR = 64

_mesh = plsc.VectorSubcoreMesh(core_axis_name="c", subcore_axis_name="s")


@functools.partial(
    pl.kernel,
    mesh=_mesh,
    compiler_params=pltpu.CompilerParams(needs_layout_passes=False),
    out_type=[
        jax.ShapeDtypeStruct((NPAD, D), jnp.float32),
        jax.ShapeDtypeStruct((NPAD, D), jnp.float32),
    ],
    scratch_types=[
        pltpu.VMEM((BLK,), jnp.int32),
        pltpu.VMEM((BLK,), jnp.int32),
        pltpu.VMEM((G,), jnp.int32),
        pltpu.VMEM((G,), jnp.int32),
        pltpu.VMEM((G, D), jnp.float32),
        pltpu.VMEM((G, D), jnp.float32),
        pltpu.VMEM((ZR, D), jnp.float32),
        pltpu.VMEM_SHARED((CHP, D), jnp.float32),
        pltpu.VMEM_SHARED((CHP, D), jnp.float32),
        pltpu.SemaphoreType.DMA,
    ],
)
def _seg_kernel(feat, srcp, dstp, ones_h, zrow_h, out_s, out_c,
                src_v, dst_v, idx_v, ldst_v, rows_v, ones_v, zrow_v,
                acc_sp, cnt_sp, sem):
    cid = lax.axis_index("c")
    sid = lax.axis_index("s")
    ebase = sid * EPT
    zb = sid * (CH // NTILE)
    wb = sid * (CH // NTILE)

    pltpu.sync_copy(ones_h, ones_v)
    pltpu.sync_copy(zrow_h, zrow_v)

    def flush():
        pltpu.async_copy(feat.at[idx_v], rows_v, sem).wait()
        pltpu.sync_copy(rows_v, acc_sp.at[ldst_v], add=True)
        pltpu.sync_copy(ones_v, cnt_sp.at[ldst_v], add=True)

    for j in range((NCHUNK + 1) // 2):
        cglob = 2 * j + cid
        live = cglob < NCHUNK

        for t in range(CH // NTILE // ZR):
            pltpu.sync_copy(zrow_v, acc_sp.at[pl.ds(zb + t * ZR, ZR)])
            pltpu.sync_copy(zrow_v, cnt_sp.at[pl.ds(zb + t * ZR, ZR)])
        plsc.subcore_barrier()

        def grp_step(g, carry, cglob=cglob):
            for u in range(G // 16):
                off = g * G + u * 16
                dvec = dst_v[pl.ds(off, 16)]
                svec = src_v[pl.ds(off, 16)]
                cvec = lax.shift_right_logical(dvec, CH_BITS)
                mask = cvec == cglob
                lvec = jnp.bitwise_and(dvec, CH - 1)
                idx_v[pl.ds(u * 16, 16)] = jnp.where(mask, svec, 0)
                ldst_v[pl.ds(u * 16, 16)] = jnp.where(
                    mask, lvec, jnp.full((16,), CH, jnp.int32) + sid)
            flush()
            return carry

        def blk_step(b, carry, grp_step=grp_step):
            pltpu.sync_copy(srcp.at[pl.ds(ebase + b * BLK, BLK)], src_v)
            pltpu.sync_copy(dstp.at[pl.ds(ebase + b * BLK, BLK)], dst_v)
            return lax.fori_loop(0, BLK // G, grp_step, carry)

        @pl.when(live)
        def _scan(blk_step=blk_step):
            lax.fori_loop(0, NB, blk_step, jnp.int32(0))

        plsc.subcore_barrier()

        @pl.when(live)
        def _writeback(cglob=cglob):
            obase = cglob * CH + wb
            for t in range(CH // NTILE // G):
                pltpu.sync_copy(acc_sp.at[pl.ds(wb + t * G, G)], rows_v)
                pltpu.sync_copy(rows_v, out_s.at[pl.ds(obase + t * G, G)])
            for t in range(CH // NTILE // G):
                pltpu.sync_copy(cnt_sp.at[pl.ds(wb + t * G, G)], rows_v)
                pltpu.sync_copy(rows_v, out_c.at[pl.ds(obase + t * G, G)])

        plsc.subcore_barrier()


RB = 512


def _fin_body(sc_ref, cc_ref, sr_ref, cr_ref, sf_ref, cf_ref,
              wc_ref, bc_ref, wr_ref, br_ref, wf_ref, bf_ref,
              hu_ref, hi_ref):
    def term(s_ref, c_ref, w_ref, b_ref):
        cnt = c_ref[:, 0:1]
        inv = 1.0 / jnp.maximum(cnt, 1.0)
        mm = jnp.dot(s_ref[:], w_ref[:], preferred_element_type=jnp.float32)
        return mm * inv + jnp.where(cnt > 0.0, b_ref[:], 0.0)

    hi_ref[:] = term(sc_ref, cc_ref, wc_ref, bc_ref)
    hu_ref[:] = (term(sr_ref, cr_ref, wr_ref, br_ref)
                 + term(sf_ref, cf_ref, wf_ref, bf_ref))


def _s_spec():
    return pl.BlockSpec((RB, D), lambda i: (i, 0))


def _c_spec():
    return pl.BlockSpec((RB, D), lambda i: (i, 0))


def _w_spec():
    return pl.BlockSpec((D, D), lambda i: (0, 0))


def _b_spec():
    return pl.BlockSpec((1, D), lambda i: (0, 0))


_fin = pl.pallas_call(
    _fin_body,
    grid=(NPAD // RB,),
    in_specs=[_s_spec(), _c_spec(), _s_spec(), _c_spec(), _s_spec(), _c_spec(),
              _w_spec(), _b_spec(), _w_spec(), _b_spec(), _w_spec(), _b_spec()],
    out_specs=[_s_spec(), _s_spec()],
    out_shape=[jax.ShapeDtypeStruct((NPAD, D), jnp.float32),
               jax.ShapeDtypeStruct((NPAD, D), jnp.float32)],
)


def _prep_edges(ei):
    src = ei[0].astype(jnp.int32)
    dst = ei[1].astype(jnp.int32)
    srcp = jnp.concatenate([src, jnp.zeros((PAD_E - E_EDGE,), jnp.int32)])
    dstp = jnp.concatenate([dst, jnp.full((PAD_E - E_EDGE,), SENT_DST, jnp.int32)])
    return srcp, dstp


def kernel(feat_user, feat_item, edge_index_user_clicks_item,
           edge_index_item_rev_clicks_user, edge_index_user_follows_user,
           W_clicks, b_clicks, W_rev, b_rev, W_follows, b_follows):
    ones_h = jnp.ones((G, D), jnp.float32)
    zrow_h = jnp.zeros((ZR, D), jnp.float32)

    sc_c, cc_c = _seg_kernel(feat_user, *_prep_edges(edge_index_user_clicks_item),
                             ones_h, zrow_h)
    sc_r, cc_r = _seg_kernel(feat_item, *_prep_edges(edge_index_item_rev_clicks_user),
                             ones_h, zrow_h)
    sc_f, cc_f = _seg_kernel(feat_user, *_prep_edges(edge_index_user_follows_user),
                             ones_h, zrow_h)

    hu, hi = _fin(sc_c, cc_c, sc_r, cc_r, sc_f, cc_f,
                  W_clicks, b_clicks.reshape(1, D), W_rev, b_rev.reshape(1, D),
                  W_follows, b_follows.reshape(1, D))
    return (hu[:N_NODE], hi[:N_NODE])


# speedup vs baseline: 80.1325x; 80.1325x over previous
import functools

import jax
import jax.numpy as jnp
from jax import lax
from jax.experimental import pallas as pl
from jax.experimental.pallas import tpu as pltpu
from jax.experimental.pallas import tpu_sc as plsc

N_NODE = 50000
E_EDGE = 600000
D = 128

CH = 5120
NCHUNK = 10
NPAD = CH * NCHUNK
CHP = CH + 128
G = 128
BLK = 2048
NTILE = 16
NB = 19
EPT = BLK * NB
PAD_E = EPT * NTILE
SENT_DST = 1 << 29
ZR = 64

_mesh = plsc.VectorSubcoreMesh(core_axis_name="c", subcore_axis_name="s")


@functools.partial(
    pl.kernel,
    mesh=_mesh,
    compiler_params=pltpu.CompilerParams(needs_layout_passes=False),
    out_type=[
        jax.ShapeDtypeStruct((NPAD, D), jnp.float32),
        jax.ShapeDtypeStruct((NPAD, D), jnp.float32),
    ],
    scratch_types=[
        pltpu.VMEM((BLK,), jnp.int32),
        pltpu.VMEM((BLK,), jnp.int32),
        pltpu.VMEM((2, G), jnp.int32),
        pltpu.VMEM((2, G), jnp.int32),
        pltpu.VMEM((G, D), jnp.float32),
        pltpu.VMEM((G, D), jnp.float32),
        pltpu.VMEM((ZR, D), jnp.float32),
        pltpu.VMEM_SHARED((CHP, D), jnp.float32),
        pltpu.VMEM_SHARED((CHP, D), jnp.float32),
        pltpu.SemaphoreType.DMA,
    ],
)
def _seg_kernel(feat, srcp, dstp, ones_h, zrow_h, out_s, out_c,
                src_v, dst_v, idx_v, ldst_v, rows_v, ones_v, zrow_v,
                acc_sp, cnt_sp, sem):
    cid = lax.axis_index("c")
    sid = lax.axis_index("s")
    ebase = sid * EPT
    zb = sid * (CH // NTILE)
    wb = sid * (CH // NTILE)

    pltpu.sync_copy(ones_h, ones_v)
    pltpu.sync_copy(zrow_h, zrow_v)

    lanes = lax.iota(jnp.int32, 16)
    splat15 = jnp.full((16, 1), 15, jnp.int32)
    _gd = lax.GatherDimensionNumbers(
        offset_dims=(), collapsed_slice_dims=(0,), start_index_map=(0,))

    def bcast15(x):
        return lax.gather(x, splat15, _gd, (1,),
                          mode=lax.GatherScatterMode.PROMISE_IN_BOUNDS)

    def memset_bufs():
        for t in range(G // 16):
            idx_v.at[0][pl.ds(t * 16, 16)] = jnp.zeros((16,), jnp.int32)
            ldst_v.at[0][pl.ds(t * 16, 16)] = lanes + (CH + t * 16)

    def flush():
        pltpu.async_copy(feat.at[idx_v.at[0]], rows_v, sem).wait()
        pltpu.sync_copy(rows_v, acc_sp.at[ldst_v.at[0]], add=True)
        pltpu.sync_copy(ones_v, cnt_sp.at[ldst_v.at[0]], add=True)
        memset_bufs()

    for j in range(NCHUNK // 2):
        cglob = 2 * j + cid
        lo = cglob * CH

        zsz = [ZR] * (CH // NTILE // ZR)
        if CH // NTILE % ZR:
            zsz.append(CH // NTILE % ZR)
        zoff = 0
        for z in zsz:
            pltpu.sync_copy(zrow_v.at[pl.ds(0, z)],
                            acc_sp.at[pl.ds(zb + zoff, z)])
            pltpu.sync_copy(zrow_v.at[pl.ds(0, z)],
                            cnt_sp.at[pl.ds(zb + zoff, z)])
            zoff += z
        memset_bufs()
        plsc.subcore_barrier()

        def vec_step(i, pos, lo=lo):
            off = i * 16
            dvec = dst_v[pl.ds(off, 16)]
            svec = src_v[pl.ds(off, 16)]
            lvec = dvec - lo
            mask = jnp.logical_and(lvec >= 0, lvec < CH)
            mi = mask.astype(jnp.int32)
            csum = jnp.cumsum(mi)
            row = 1 - mi
            col = jnp.where(mask, pos + csum - 1, lanes)
            plsc.store_scatter(idx_v, [row, col], svec)
            plsc.store_scatter(ldst_v, [row, col], lvec)
            pos = pos + bcast15(csum)
            full = jnp.any(pos > G - 16)

            @pl.when(full)
            def _():
                flush()

            return jnp.where(full, jnp.zeros((16,), jnp.int32), pos)

        def blk_step(b, pos, vec_step=vec_step):
            pltpu.sync_copy(srcp.at[pl.ds(ebase + b * BLK, BLK)], src_v)
            pltpu.sync_copy(dstp.at[pl.ds(ebase + b * BLK, BLK)], dst_v)
            return lax.fori_loop(0, BLK // 16, vec_step, pos)

        lax.fori_loop(0, NB, blk_step, jnp.zeros((16,), jnp.int32))
        flush()
        plsc.subcore_barrier()

        obase = cglob * CH + wb
        woff = 0
        for z in zsz:
            pltpu.sync_copy(acc_sp.at[pl.ds(wb + woff, z)],
                            rows_v.at[pl.ds(0, z)])
            pltpu.sync_copy(rows_v.at[pl.ds(0, z)],
                            out_s.at[pl.ds(obase + woff, z)])
            woff += z
        woff = 0
        for z in zsz:
            pltpu.sync_copy(cnt_sp.at[pl.ds(wb + woff, z)],
                            rows_v.at[pl.ds(0, z)])
            pltpu.sync_copy(rows_v.at[pl.ds(0, z)],
                            out_c.at[pl.ds(obase + woff, z)])
            woff += z
        plsc.subcore_barrier()


RB = 512


def _fin_body(sc_ref, cc_ref, sr_ref, cr_ref, sf_ref, cf_ref,
              wc_ref, bc_ref, wr_ref, br_ref, wf_ref, bf_ref,
              hu_ref, hi_ref):
    def term(s_ref, c_ref, w_ref, b_ref):
        cnt = c_ref[:, 0:1]
        inv = 1.0 / jnp.maximum(cnt, 1.0)
        mm = jnp.dot(s_ref[:], w_ref[:], preferred_element_type=jnp.float32)
        return mm * inv + jnp.where(cnt > 0.0, b_ref[:], 0.0)

    hi_ref[:] = term(sc_ref, cc_ref, wc_ref, bc_ref)
    hu_ref[:] = (term(sr_ref, cr_ref, wr_ref, br_ref)
                 + term(sf_ref, cf_ref, wf_ref, bf_ref))


def _s_spec():
    return pl.BlockSpec((RB, D), lambda i: (i, 0))


def _c_spec():
    return pl.BlockSpec((RB, D), lambda i: (i, 0))


def _w_spec():
    return pl.BlockSpec((D, D), lambda i: (0, 0))


def _b_spec():
    return pl.BlockSpec((1, D), lambda i: (0, 0))


_fin = pl.pallas_call(
    _fin_body,
    grid=(NPAD // RB,),
    in_specs=[_s_spec(), _c_spec(), _s_spec(), _c_spec(), _s_spec(), _c_spec(),
              _w_spec(), _b_spec(), _w_spec(), _b_spec(), _w_spec(), _b_spec()],
    out_specs=[_s_spec(), _s_spec()],
    out_shape=[jax.ShapeDtypeStruct((NPAD, D), jnp.float32),
               jax.ShapeDtypeStruct((NPAD, D), jnp.float32)],
)


def _prep_edges(ei):
    src = ei[0].astype(jnp.int32)
    dst = ei[1].astype(jnp.int32)
    srcp = jnp.concatenate([src, jnp.zeros((PAD_E - E_EDGE,), jnp.int32)])
    dstp = jnp.concatenate([dst, jnp.full((PAD_E - E_EDGE,), SENT_DST, jnp.int32)])
    return srcp, dstp


def kernel(feat_user, feat_item, edge_index_user_clicks_item,
           edge_index_item_rev_clicks_user, edge_index_user_follows_user,
           W_clicks, b_clicks, W_rev, b_rev, W_follows, b_follows):
    ones_h = jnp.ones((G, D), jnp.float32)
    zrow_h = jnp.zeros((ZR, D), jnp.float32)

    sc_c, cc_c = _seg_kernel(feat_user, *_prep_edges(edge_index_user_clicks_item),
                             ones_h, zrow_h)
    sc_r, cc_r = _seg_kernel(feat_item, *_prep_edges(edge_index_item_rev_clicks_user),
                             ones_h, zrow_h)
    sc_f, cc_f = _seg_kernel(feat_user, *_prep_edges(edge_index_user_follows_user),
                             ones_h, zrow_h)

    hu, hi = _fin(sc_c, cc_c, sc_r, cc_r, sc_f, cc_f,
                  W_clicks, b_clicks.reshape(1, D), W_rev, b_rev.reshape(1, D),
                  W_follows, b_follows.reshape(1, D))
    return (hu[:N_NODE], hi[:N_NODE])


# speedup vs baseline: 113.6166x; 1.4179x over previous
import functools

import jax
import jax.numpy as jnp
from jax import lax
from jax.experimental import pallas as pl
from jax.experimental.pallas import tpu as pltpu
from jax.experimental.pallas import tpu_sc as plsc

N_NODE = 50000
E_EDGE = 600000
D = 128

CH = 4224
NCHUNK = 12
NPAD = CH * NCHUNK
CHP = CH + 64
NR = 2
GE = NR * 128
BLK = 2048
NTILE = 16
NB = 19
EPT = BLK * NB
PAD_E = EPT * NTILE
SENT_DST = 1 << 29
ZR = 64

_mesh = plsc.VectorSubcoreMesh(core_axis_name="c", subcore_axis_name="s")


@functools.partial(
    pl.kernel,
    mesh=_mesh,
    compiler_params=pltpu.CompilerParams(needs_layout_passes=False),
    out_type=[
        jax.ShapeDtypeStruct((NPAD, D), jnp.float32),
        jax.ShapeDtypeStruct((NPAD, D), jnp.float32),
    ],
    scratch_types=[
        pltpu.VMEM((BLK,), jnp.int32),
        pltpu.VMEM((BLK,), jnp.int32),
        pltpu.VMEM((NR + 1, 128), jnp.int32),
        pltpu.VMEM((NR + 1, 128), jnp.int32),
        pltpu.VMEM((GE, D), jnp.float32),
        pltpu.VMEM((128, D), jnp.float32),
        pltpu.VMEM((ZR, D), jnp.float32),
        pltpu.VMEM_SHARED((CHP, D), jnp.float32),
        pltpu.VMEM_SHARED((CHP, D), jnp.float32),
        pltpu.SemaphoreType.DMA,
        pltpu.SemaphoreType.DMA,
    ],
)
def _seg_kernel(feat, srcp, dstp, ones_h, zrow_h, out_s, out_c,
                src_v, dst_v, idx_v, ldst_v, rows_v, ones_v, zrow_v,
                acc_sp, cnt_sp, sem, sem2):
    cid = lax.axis_index("c")
    sid = lax.axis_index("s")
    ebase = sid * EPT
    zb = sid * (CH // NTILE)
    wb = sid * (CH // NTILE)

    pltpu.sync_copy(ones_h, ones_v)
    pltpu.sync_copy(zrow_h, zrow_v)

    lanes = lax.iota(jnp.int32, 16)
    splat15 = jnp.full((16, 1), 15, jnp.int32)
    _gd = lax.GatherDimensionNumbers(
        offset_dims=(), collapsed_slice_dims=(0,), start_index_map=(0,))

    def bcast15(x):
        return lax.gather(x, splat15, _gd, (1,),
                          mode=lax.GatherScatterMode.PROMISE_IN_BOUNDS)

    def memset_bufs():
        for t in range(NR):
            for u in range(8):
                idx_v.at[t][pl.ds(u * 16, 16)] = jnp.zeros((16,), jnp.int32)
                ldst_v.at[t][pl.ds(u * 16, 16)] = lanes + (CH + (u % 4) * 16)

    def flush():
        gp = [pltpu.async_copy(feat.at[idx_v.at[t]],
                               rows_v.at[pl.ds(t * 128, 128)], sem)
              for t in range(NR)]
        for g in gp:
            g.wait()
        for t in range(NR):
            pltpu.sync_copy(rows_v.at[pl.ds(t * 128, 128)],
                            acc_sp.at[ldst_v.at[t]], add=True)
            pltpu.sync_copy(ones_v, cnt_sp.at[ldst_v.at[t]], add=True)
        memset_bufs()

    def chunk_body(j, chunk_carry):
        cglob = 2 * j + cid
        lo = cglob * CH

        zsz = [ZR] * (CH // NTILE // ZR)
        if CH // NTILE % ZR:
            zsz.append(CH // NTILE % ZR)
        zoff = 0
        for z in zsz:
            pltpu.sync_copy(zrow_v.at[pl.ds(0, z)],
                            acc_sp.at[pl.ds(zb + zoff, z)])
            pltpu.sync_copy(zrow_v.at[pl.ds(0, z)],
                            cnt_sp.at[pl.ds(zb + zoff, z)])
            zoff += z
        memset_bufs()
        plsc.subcore_barrier()

        def vec_step(i, pos, lo=lo):
            off = i * 16
            dvec = dst_v[pl.ds(off, 16)]
            svec = src_v[pl.ds(off, 16)]
            lvec = dvec - lo
            mask = jnp.logical_and(lvec >= 0, lvec < CH)
            mi = mask.astype(jnp.int32)
            csum = jnp.cumsum(mi)
            p = pos + csum - 1
            rowi = jnp.where(mask, lax.shift_right_logical(p, 7), NR)
            coli = jnp.where(mask, jnp.bitwise_and(p, 127), lanes)
            plsc.store_scatter(idx_v, [rowi, coli], svec)
            plsc.store_scatter(ldst_v, [rowi, coli], lvec)
            pos = pos + bcast15(csum)
            full = jnp.any(pos > GE - 16)

            @pl.when(full)
            def _():
                flush()

            return jnp.where(full, jnp.zeros((16,), jnp.int32), pos)

        def blk_step(b, pos, vec_step=vec_step):
            pltpu.sync_copy(srcp.at[pl.ds(ebase + b * BLK, BLK)], src_v)
            pltpu.sync_copy(dstp.at[pl.ds(ebase + b * BLK, BLK)], dst_v)
            return lax.fori_loop(0, BLK // 16, vec_step, pos)

        lax.fori_loop(0, NB, blk_step, jnp.zeros((16,), jnp.int32))
        flush()
        plsc.subcore_barrier()

        obase = cglob * CH + wb
        woff = 0
        for z in zsz:
            pltpu.sync_copy(acc_sp.at[pl.ds(wb + woff, z)],
                            rows_v.at[pl.ds(0, z)])
            pltpu.sync_copy(rows_v.at[pl.ds(0, z)],
                            out_s.at[pl.ds(obase + woff, z)])
            woff += z
        woff = 0
        for z in zsz:
            pltpu.sync_copy(cnt_sp.at[pl.ds(wb + woff, z)],
                            rows_v.at[pl.ds(0, z)])
            pltpu.sync_copy(rows_v.at[pl.ds(0, z)],
                            out_c.at[pl.ds(obase + woff, z)])
            woff += z
        plsc.subcore_barrier()
        return chunk_carry

    lax.fori_loop(0, NCHUNK // 2, chunk_body, jnp.int32(0))


RB = 512


def _fin_body(sc_ref, cc_ref, sr_ref, cr_ref, sf_ref, cf_ref,
              wc_ref, bc_ref, wr_ref, br_ref, wf_ref, bf_ref,
              hu_ref, hi_ref):
    def term(s_ref, c_ref, w_ref, b_ref):
        cnt = c_ref[:, 0:1]
        inv = 1.0 / jnp.maximum(cnt, 1.0)
        mm = jnp.dot(s_ref[:], w_ref[:], preferred_element_type=jnp.float32)
        return mm * inv + jnp.where(cnt > 0.0, b_ref[:], 0.0)

    hi_ref[:] = term(sc_ref, cc_ref, wc_ref, bc_ref)
    hu_ref[:] = (term(sr_ref, cr_ref, wr_ref, br_ref)
                 + term(sf_ref, cf_ref, wf_ref, bf_ref))


def _s_spec():
    return pl.BlockSpec((RB, D), lambda i: (i, 0))


def _w_spec():
    return pl.BlockSpec((D, D), lambda i: (0, 0))


def _b_spec():
    return pl.BlockSpec((1, D), lambda i: (0, 0))


_fin = pl.pallas_call(
    _fin_body,
    grid=(NPAD // RB,),
    in_specs=[_s_spec(), _s_spec(), _s_spec(), _s_spec(), _s_spec(), _s_spec(),
              _w_spec(), _b_spec(), _w_spec(), _b_spec(), _w_spec(), _b_spec()],
    out_specs=[_s_spec(), _s_spec()],
    out_shape=[jax.ShapeDtypeStruct((NPAD, D), jnp.float32),
               jax.ShapeDtypeStruct((NPAD, D), jnp.float32)],
)


def _prep_edges(ei):
    src = ei[0].astype(jnp.int32)
    dst = ei[1].astype(jnp.int32)
    srcp = jnp.concatenate([src, jnp.zeros((PAD_E - E_EDGE,), jnp.int32)])
    dstp = jnp.concatenate([dst, jnp.full((PAD_E - E_EDGE,), SENT_DST, jnp.int32)])
    return srcp, dstp


def kernel(feat_user, feat_item, edge_index_user_clicks_item,
           edge_index_item_rev_clicks_user, edge_index_user_follows_user,
           W_clicks, b_clicks, W_rev, b_rev, W_follows, b_follows):
    ones_h = jnp.ones((128, D), jnp.float32)
    zrow_h = jnp.zeros((ZR, D), jnp.float32)

    sc_c, cc_c = _seg_kernel(feat_user, *_prep_edges(edge_index_user_clicks_item),
                             ones_h, zrow_h)
    dep1 = zrow_h + 0.0 * sc_c[0, 0]
    sc_r, cc_r = _seg_kernel(feat_item, *_prep_edges(edge_index_item_rev_clicks_user),
                             ones_h, dep1)
    dep2 = zrow_h + 0.0 * sc_r[0, 0]
    sc_f, cc_f = _seg_kernel(feat_user, *_prep_edges(edge_index_user_follows_user),
                             ones_h, dep2)

    hu, hi = _fin(sc_c, cc_c, sc_r, cc_r, sc_f, cc_f,
                  W_clicks, b_clicks.reshape(1, D), W_rev, b_rev.reshape(1, D),
                  W_follows, b_follows.reshape(1, D))
    return (hu[:N_NODE], hi[:N_NODE])
